# Initial kernel scaffold; baseline (speedup 1.0000x reference)
import jax
import jax.numpy as jnp
from jax.experimental import pallas as pl


def kernel(x, edge_index, W1, b1, W2, b2):
    raise NotImplementedError("write your pallas kernel here")


# speedup vs baseline: 82.0941x; 82.0941x over previous
import functools

import jax
import jax.numpy as jnp
from jax import lax
from jax.experimental import pallas as pl
from jax.experimental.pallas import tpu as pltpu
from jax.experimental.pallas import tpu_sc as plsc

NC = 2
NS = 16
NW = NC * NS

SUB = 128
NSUB = 16
KB = SUB * NSUB


def _mesh():
    return plsc.VectorSubcoreMesh(core_axis_name="c", subcore_axis_name="s",
                                  num_cores=NC, num_subcores=NS)


def _worker(c, s):
    return s * NC + c


def _batch_base(wid, nbatch):
    hi = nbatch // NW + 1
    lo = nbatch // NW
    rem = nbatch - lo * NW
    nb = jnp.where(wid < rem, hi, lo)
    base = jnp.where(wid < rem, wid * hi, rem * hi + (wid - rem) * lo)
    return base, nb


def _deg_body(n_pad, nbatch, dst3, degp, acc, didx, ones, vbuf, sems):
    c = lax.axis_index("c")
    s = lax.axis_index("s")
    wid = _worker(c, s)
    rt = n_pad // NS
    r0 = s * rt

    for i in range(SUB // 16):
        ones[pl.ds(i * 16, 16)] = jnp.full((16,), 1.0, jnp.float32)

    def zb(i, _):
        vbuf[pl.ds(i * 16, 16)] = jnp.zeros((16,), jnp.float32)
        return 0

    lax.fori_loop(0, rt // 16, zb, 0)
    pltpu.sync_copy(vbuf, acc.at[pl.ds(r0, rt)])
    plsc.subcore_barrier()

    base, nb = _batch_base(wid, nbatch)

    def body(k, _):
        b = base + k
        pltpu.sync_copy(dst3.at[b], didx)
        descs = [
            pltpu.async_copy(ones, acc.at[didx.at[j]], sems, add=True)
            for j in range(NSUB)
        ]
        for d in descs:
            d.wait()
        return 0

    lax.fori_loop(0, nb, body, 0)
    plsc.subcore_barrier()

    pltpu.sync_copy(acc.at[pl.ds(r0, rt)], vbuf)
    pltpu.sync_copy(vbuf, degp.at[pl.ds(c * n_pad + r0, rt)])


_SC_PARAMS = pltpu.CompilerParams(use_tc_tiling_on_sc=False)


def _deg_call(n_pad, nbatch, dst3):
    body = functools.partial(_deg_body, n_pad, nbatch)
    return pl.kernel(
        body,
        out_type=jax.ShapeDtypeStruct((NC * n_pad,), jnp.float32),
        mesh=_mesh(),
        compiler_params=_SC_PARAMS,
        scratch_types=[
            pltpu.VMEM_SHARED((n_pad,), jnp.float32),
            pltpu.VMEM((NSUB, SUB), jnp.int32),
            pltpu.VMEM((SUB,), jnp.float32),
            pltpu.VMEM((n_pad // NS,), jnp.float32),
            pltpu.SemaphoreType.DMA,
        ],
    )(dst3)


def _scat_body(n_pad, nbatch, d, table, src3, dst3, zrows, accp, acc,
               sidx, didx, rows, semg, sems):
    c = lax.axis_index("c")
    s = lax.axis_index("s")
    wid = _worker(c, s)
    rt = n_pad // NS
    r0 = s * rt

    pltpu.sync_copy(zrows, rows)
    nfull = rt // KB
    tail = rt - nfull * KB
    for ch in range(nfull):
        pltpu.sync_copy(rows, acc.at[pl.ds(r0 + ch * KB, KB)])
    if tail:
        pltpu.sync_copy(rows.at[pl.ds(0, tail)],
                        acc.at[pl.ds(r0 + nfull * KB, tail)])
    plsc.subcore_barrier()

    base, nb = _batch_base(wid, nbatch)

    def body(k, _):
        b = base + k
        pltpu.sync_copy(src3.at[b], sidx)
        pltpu.sync_copy(dst3.at[b], didx)
        g = [
            pltpu.async_copy(table.at[sidx.at[j]],
                             rows.at[pl.ds(j * SUB, SUB)], semg)
            for j in range(NSUB)
        ]
        for dsc in g:
            dsc.wait()
        sc = [
            pltpu.async_copy(rows.at[pl.ds(j * SUB, SUB)],
                             acc.at[didx.at[j]], sems, add=True)
            for j in range(NSUB)
        ]
        for dsc in sc:
            dsc.wait()
        return 0

    lax.fori_loop(0, nb, body, 0)
    plsc.subcore_barrier()

    for ch in range(nfull):
        pltpu.sync_copy(acc.at[pl.ds(r0 + ch * KB, KB)], rows)
        pltpu.sync_copy(rows, accp.at[pl.ds(c * n_pad + r0 + ch * KB, KB)])
    if tail:
        pltpu.sync_copy(acc.at[pl.ds(r0 + nfull * KB, tail)],
                        rows.at[pl.ds(0, tail)])
        pltpu.sync_copy(rows.at[pl.ds(0, tail)],
                        accp.at[pl.ds(c * n_pad + r0 + nfull * KB, tail)])


def _scat_call(n_pad, nbatch, d, table, src3, dst3):
    body = functools.partial(_scat_body, n_pad, nbatch, d)
    zrows = jnp.zeros((KB, d), jnp.float32)
    return pl.kernel(
        body,
        out_type=jax.ShapeDtypeStruct((NC * n_pad, d), jnp.float32),
        mesh=_mesh(),
        compiler_params=_SC_PARAMS,
        scratch_types=[
            pltpu.VMEM_SHARED((n_pad, d), jnp.float32),
            pltpu.VMEM((NSUB, SUB), jnp.int32),
            pltpu.VMEM((NSUB, SUB), jnp.int32),
            pltpu.VMEM((KB, d), jnp.float32),
            pltpu.SemaphoreType.DMA,
            pltpu.SemaphoreType.DMA,
        ],
    )(table, src3, dst3, zrows)


BTC = 4096


def _dense1_body(x, d0, d1, w1, h1s):
    dis = lax.rsqrt(d0[...] + d1[...] + 1.0)
    xr = x[...]
    w = w1[...]
    h = (xr[:, 0:1] * w[0:1, :] + xr[:, 1:2] * w[1:2, :]
         + xr[:, 2:3] * w[2:3, :])
    h1s[...] = dis * h


def _dense2_body(a0, a1, h1s, d0, d1, w2, b1, h2s):
    dis = lax.rsqrt(d0[...] + d1[...] + 1.0)
    z = dis * (a0[...] + a1[...] + h1s[...]) + b1[...]
    h = jnp.maximum(z, 0.0)
    w = w2[...]
    h2 = h[:, 0:1] * w[0:1, :]
    for k in range(1, 8):
        h2 = h2 + h[:, k:k + 1] * w[k:k + 1, :]
    h2s[...] = dis * h2


def _dense3_body(a0, a1, h2s, d0, d1, b2, out):
    dis = lax.rsqrt(d0[...] + d1[...] + 1.0)
    out[...] = dis * (a0[...] + a1[...] + h2s[...]) + b2[...]


def _row_spec(d):
    return pl.BlockSpec((BTC, d), lambda i: (i, 0))


def _full_spec(shape):
    return pl.BlockSpec(shape, lambda i: tuple(0 for _ in shape))


def _dense1(n, x, d0, d1, w1):
    grid = (pl.cdiv(n, BTC),)
    return pl.pallas_call(
        _dense1_body,
        grid=grid,
        in_specs=[_row_spec(3), _row_spec(1), _row_spec(1), _full_spec((3, 8))],
        out_specs=_row_spec(8),
        out_shape=jax.ShapeDtypeStruct((n, 8), jnp.float32),
    )(x, d0, d1, w1)


def _dense2(n, a0, a1, h1s, d0, d1, w2, b1):
    grid = (pl.cdiv(n, BTC),)
    return pl.pallas_call(
        _dense2_body,
        grid=grid,
        in_specs=[_row_spec(8), _row_spec(8), _row_spec(8), _row_spec(1),
                  _row_spec(1), _full_spec((8, 2)), _full_spec((1, 8))],
        out_specs=_row_spec(2),
        out_shape=jax.ShapeDtypeStruct((n, 2), jnp.float32),
    )(a0, a1, h1s, d0, d1, w2, b1)


def _dense3(n, a0, a1, h2s, d0, d1, b2):
    grid = (pl.cdiv(n, BTC),)
    return pl.pallas_call(
        _dense3_body,
        grid=grid,
        in_specs=[_row_spec(2), _row_spec(2), _row_spec(2), _row_spec(1),
                  _row_spec(1), _full_spec((1, 2))],
        out_specs=_row_spec(2),
        out_shape=jax.ShapeDtypeStruct((n, 2), jnp.float32),
    )(a0, a1, h2s, d0, d1, b2)


def kernel(x, edge_index, W1, b1, W2, b2):
    n = x.shape[0]
    e = edge_index.shape[1]
    assert e % KB == 0
    nbatch = e // KB
    n_pad = ((n + (NS * 8) - 1) // (NS * 8)) * (NS * 8)
    if (n_pad // NS) % 16:
        n_pad = ((n_pad + NS * 16 - 1) // (NS * 16)) * (NS * 16)

    src3 = edge_index[0].reshape(nbatch, NSUB, SUB)
    dst3 = edge_index[1].reshape(nbatch, NSUB, SUB)

    def pad_rows(t):
        return jnp.concatenate(
            [t, jnp.zeros((n_pad - n, t.shape[1]), t.dtype)], axis=0)

    degp = _deg_call(n_pad, nbatch, dst3)
    d0 = degp[:n].reshape(n, 1)
    d1 = degp[n_pad:n_pad + n].reshape(n, 1)

    h1s = _dense1(n, x, d0, d1, W1)

    acc1 = _scat_call(n_pad, nbatch, 8, pad_rows(h1s), src3, dst3)
    a10 = acc1[:n]
    a11 = acc1[n_pad:n_pad + n]

    h2s = _dense2(n, a10, a11, h1s, d0, d1, W2, b1.reshape(1, 8))

    acc2 = _scat_call(n_pad, nbatch, 2, pad_rows(h2s), src3, dst3)
    a20 = acc2[:n]
    a21 = acc2[n_pad:n_pad + n]

    return _dense3(n, a20, a21, h2s, d0, d1, b2.reshape(1, 2))


# speedup vs baseline: 83.9146x; 1.0222x over previous
import functools

import jax
import jax.numpy as jnp
from jax import lax
from jax.experimental import pallas as pl
from jax.experimental.pallas import tpu as pltpu
from jax.experimental.pallas import tpu_sc as plsc

NC = 2
NS = 16
NW = NC * NS

SUB = 128
NSUB = 16
KB = SUB * NSUB


def _mesh():
    return plsc.VectorSubcoreMesh(core_axis_name="c", subcore_axis_name="s",
                                  num_cores=NC, num_subcores=NS)


_SC_PARAMS = pltpu.CompilerParams(use_tc_tiling_on_sc=False)


def _worker(c, s):
    return s * NC + c


def _batch_base(wid, nbatch):
    hi = nbatch // NW + 1
    lo = nbatch // NW
    rem = nbatch - lo * NW
    nb = jnp.where(wid < rem, hi, lo)
    base = jnp.where(wid < rem, wid * hi, rem * hi + (wid - rem) * lo)
    return base, nb


def _deg_body(n_pad, nbatch, dst3, degp, acc, didx, ones, vbuf, sems):
    c = lax.axis_index("c")
    s = lax.axis_index("s")
    wid = _worker(c, s)
    rt = n_pad // NS
    r0 = s * rt

    for i in range(SUB // 16):
        ones[pl.ds(i * 16, 16)] = jnp.full((16,), 1.0, jnp.float32)

    def zb(i, _):
        vbuf[pl.ds(i * 16, 16)] = jnp.zeros((16,), jnp.float32)
        return 0

    lax.fori_loop(0, rt // 16, zb, 0)
    pltpu.sync_copy(vbuf, acc.at[pl.ds(r0, rt)])
    plsc.subcore_barrier()

    base, nb = _batch_base(wid, nbatch)

    def body(k, _):
        b = base + k
        pltpu.sync_copy(dst3.at[b], didx)
        descs = [
            pltpu.async_copy(ones, acc.at[didx.at[j]], sems, add=True)
            for j in range(NSUB)
        ]
        for dsc in descs:
            dsc.wait()
        return 0

    lax.fori_loop(0, nb, body, 0)
    plsc.subcore_barrier()

    pltpu.sync_copy(acc.at[pl.ds(r0, rt)], vbuf)
    pltpu.sync_copy(vbuf, degp.at[pl.ds(c * n_pad + r0, rt)])


def _deg_call(n_pad, nbatch, dst3):
    body = functools.partial(_deg_body, n_pad, nbatch)
    return pl.kernel(
        body,
        out_type=jax.ShapeDtypeStruct((NC * n_pad,), jnp.float32),
        mesh=_mesh(),
        compiler_params=_SC_PARAMS,
        scratch_types=[
            pltpu.VMEM_SHARED((n_pad,), jnp.float32),
            pltpu.VMEM((NSUB, SUB), jnp.int32),
            pltpu.VMEM((SUB,), jnp.float32),
            pltpu.VMEM((n_pad // NS,), jnp.float32),
            pltpu.SemaphoreType.DMA,
        ],
    )(dst3)


def _scat_body(n_pad, nbatch, d, table, src3, dst3, zrows, accp, acc,
               sidx, didx, rows, semg, sems):
    c = lax.axis_index("c")
    s = lax.axis_index("s")
    wid = _worker(c, s)
    rt = n_pad // NS
    r0 = s * rt

    pltpu.sync_copy(zrows, rows)
    nfull = rt // KB
    tail = rt - nfull * KB
    for ch in range(nfull):
        pltpu.sync_copy(rows, acc.at[pl.ds(r0 + ch * KB, KB)])
    if tail:
        pltpu.sync_copy(rows.at[pl.ds(0, tail)],
                        acc.at[pl.ds(r0 + nfull * KB, tail)])
    plsc.subcore_barrier()

    base, nb = _batch_base(wid, nbatch)

    def body(k, _):
        b = base + k
        pltpu.sync_copy(src3.at[b], sidx)
        pltpu.sync_copy(dst3.at[b], didx)
        g = [
            pltpu.async_copy(table.at[sidx.at[j]],
                             rows.at[pl.ds(j * SUB, SUB)], semg)
            for j in range(NSUB)
        ]
        for dsc in g:
            dsc.wait()
        sc = [
            pltpu.async_copy(rows.at[pl.ds(j * SUB, SUB)],
                             acc.at[didx.at[j]], sems, add=True)
            for j in range(NSUB)
        ]
        for dsc in sc:
            dsc.wait()
        return 0

    lax.fori_loop(0, nb, body, 0)
    plsc.subcore_barrier()

    for ch in range(nfull):
        pltpu.sync_copy(acc.at[pl.ds(r0 + ch * KB, KB)], rows)
        pltpu.sync_copy(rows, accp.at[pl.ds(c * n_pad + r0 + ch * KB, KB)])
    if tail:
        pltpu.sync_copy(acc.at[pl.ds(r0 + nfull * KB, tail)],
                        rows.at[pl.ds(0, tail)])
        pltpu.sync_copy(rows.at[pl.ds(0, tail)],
                        accp.at[pl.ds(c * n_pad + r0 + nfull * KB, tail)])


def _scat_call(n_pad, nbatch, d, table, src3, dst3):
    body = functools.partial(_scat_body, n_pad, nbatch, d)
    zrows = jnp.zeros((KB, d), jnp.float32)
    return pl.kernel(
        body,
        out_type=jax.ShapeDtypeStruct((NC * n_pad, d), jnp.float32),
        mesh=_mesh(),
        compiler_params=_SC_PARAMS,
        scratch_types=[
            pltpu.VMEM_SHARED((n_pad, d), jnp.float32),
            pltpu.VMEM((NSUB, SUB), jnp.int32),
            pltpu.VMEM((NSUB, SUB), jnp.int32),
            pltpu.VMEM((KB, d), jnp.float32),
            pltpu.SemaphoreType.DMA,
            pltpu.SemaphoreType.DMA,
        ],
    )(table, src3, dst3, zrows)


BTC = 4096


def _dense1_body(x, d0, d1, w1, h1s):
    dis = 1.0 / jnp.sqrt(d0[...] + d1[...] + 1.0)
    xr = x[...].astype(jnp.bfloat16).astype(jnp.float32)
    w = w1[...].astype(jnp.bfloat16).astype(jnp.float32)
    h = (xr[:, 0:1] * w[0:1, :] + xr[:, 1:2] * w[1:2, :]
         + xr[:, 2:3] * w[2:3, :])
    h1s[...] = dis * h


def _dense2_body(a0, a1, h1s, d0, d1, w2, b1, h2s):
    dis = 1.0 / jnp.sqrt(d0[...] + d1[...] + 1.0)
    z = dis * (a0[...] + a1[...] + h1s[...]) + b1[...]
    h = jnp.maximum(z, 0.0).astype(jnp.bfloat16).astype(jnp.float32)
    w = w2[...].astype(jnp.bfloat16).astype(jnp.float32)
    h2 = h[:, 0:1] * w[0:1, :]
    for k in range(1, 8):
        h2 = h2 + h[:, k:k + 1] * w[k:k + 1, :]
    h2s[...] = dis * h2


def _dense3_body(a0, a1, h2s, d0, d1, b2, out):
    dis = 1.0 / jnp.sqrt(d0[...] + d1[...] + 1.0)
    out[...] = (dis * (a0[...] + a1[...] + h2s[...]))[:, 0:2] + b2[...]


def _row_spec(d):
    return pl.BlockSpec((BTC, d), lambda i: (i, 0))


def _full_spec(shape):
    return pl.BlockSpec(shape, lambda i: tuple(0 for _ in shape))


def _dense1(n, x, d0, d1, w1):
    grid = (pl.cdiv(n, BTC),)
    return pl.pallas_call(
        _dense1_body,
        grid=grid,
        in_specs=[_row_spec(3), _row_spec(1), _row_spec(1), _full_spec((3, 8))],
        out_specs=_row_spec(8),
        out_shape=jax.ShapeDtypeStruct((n, 8), jnp.float32),
    )(x, d0, d1, w1)


def _dense2(n, a0, a1, h1s, d0, d1, w2, b1):
    grid = (pl.cdiv(n, BTC),)
    return pl.pallas_call(
        _dense2_body,
        grid=grid,
        in_specs=[_row_spec(8), _row_spec(8), _row_spec(8), _row_spec(1),
                  _row_spec(1), _full_spec((8, 8)), _full_spec((1, 8))],
        out_specs=_row_spec(8),
        out_shape=jax.ShapeDtypeStruct((n, 8), jnp.float32),
    )(a0, a1, h1s, d0, d1, w2, b1)


def _dense3(n, a0, a1, h2s, d0, d1, b2):
    grid = (pl.cdiv(n, BTC),)
    return pl.pallas_call(
        _dense3_body,
        grid=grid,
        in_specs=[_row_spec(8), _row_spec(8), _row_spec(8), _row_spec(1),
                  _row_spec(1), _full_spec((1, 2))],
        out_specs=_row_spec(2),
        out_shape=jax.ShapeDtypeStruct((n, 2), jnp.float32),
    )(a0, a1, h2s, d0, d1, b2)


def kernel(x, edge_index, W1, b1, W2, b2):
    n = x.shape[0]
    e = edge_index.shape[1]
    assert e % KB == 0
    nbatch = e // KB
    n_pad = ((n + (NS * 8) - 1) // (NS * 8)) * (NS * 8)
    if (n_pad // NS) % 16:
        n_pad = ((n_pad + NS * 16 - 1) // (NS * 16)) * (NS * 16)

    src3 = edge_index[0].reshape(nbatch, NSUB, SUB)
    dst3 = edge_index[1].reshape(nbatch, NSUB, SUB)

    def pad_rows(t):
        return jnp.concatenate(
            [t, jnp.zeros((n_pad - n, t.shape[1]), t.dtype)], axis=0)

    degp = _deg_call(n_pad, nbatch, dst3)
    d0 = degp[:n].reshape(n, 1)
    d1 = degp[n_pad:n_pad + n].reshape(n, 1)

    h1s = _dense1(n, x, d0, d1, W1)

    acc1 = _scat_call(n_pad, nbatch, 8, pad_rows(h1s), src3, dst3)
    a10 = acc1[:n]
    a11 = acc1[n_pad:n_pad + n]

    W2p = jnp.concatenate([W2, jnp.zeros((8, 6), jnp.float32)], axis=1)
    h2s = _dense2(n, a10, a11, h1s, d0, d1, W2p, b1.reshape(1, 8))

    acc2 = _scat_call(n_pad, nbatch, 8, pad_rows(h2s), src3, dst3)
    a20 = acc2[:n]
    a21 = acc2[n_pad:n_pad + n]

    return _dense3(n, a20, a21, h2s, d0, d1, b2.reshape(1, 2))
